# baseline (speedup 1.0000x reference)
import functools

import jax
import jax.numpy as jnp
from jax import lax
from jax.experimental import pallas as pl
from jax.experimental.pallas import tpu as pltpu
from jax.experimental.pallas import tpu_sc as plsc


def _make_kernel(B, L, D, NC, NS):
    NW = NC * NS
    BPW = B // NW
    CH = 40
    SPLIT = L // CH
    NCHUNK = BPW * SPLIT
    NBUF = 4
    NP = 2
    W = 4 * D
    starts = list(range(0, CH - 15, 16))
    if starts[-1] != CH - 16:
        starts.append(CH - 16)
    mesh = plsc.VectorSubcoreMesh(core_axis_name="c", subcore_axis_name="s")

    @functools.partial(
        pl.kernel,
        mesh=mesh,
        out_type=jax.ShapeDtypeStruct((B, L, D), jnp.float32),
        scratch_types=(
            [pltpu.VMEM((NCHUNK, CH), jnp.int32),
             pltpu.VMEM((NBUF, CH), jnp.int32),
             pltpu.VMEM((NBUF, CH, W), jnp.float32),
             pltpu.VMEM((NP, CH, D), jnp.float32)]
            + [pltpu.SemaphoreType.DMA] * (NBUF + NP + 1)
        ),
        compiler_params=pltpu.CompilerParams(needs_layout_passes=False),
    )
    def k(table_hbm, x_hbm, out_hbm, idx_v, widx_v, rows_v, pack_v, *sems):
        gsem = sems[:NBUF]
        osem = sems[NBUF:NBUF + NP]
        isem = sems[NBUF + NP]
        wid = lax.axis_index("s") * NC + lax.axis_index("c")
        pltpu.async_copy(x_hbm.at[wid], idx_v, isem).wait()
        lanes = lax.iota(jnp.int32, 16)

        def compute_widx(j, m):
            for st in starts:
                iv = idx_v[j, pl.ds(st, 16)]
                widx_v[m, pl.ds(st, 16)] = iv >> 2

        def fire_gather(m, b):
            pltpu.async_copy(table_hbm.at[widx_v.at[m]], rows_v.at[b],
                             gsem[b])

        def wait_gather(b):
            pltpu.make_async_copy(table_hbm.at[widx_v.at[b]], rows_v.at[b],
                                  gsem[b]).wait()

        def out_ref(j):
            return out_hbm.at[wid * BPW + j // SPLIT,
                              pl.ds((j % SPLIT) * CH, CH)]

        def fire_out(j, p):
            pltpu.async_copy(pack_v.at[p], out_ref(j), osem[p])

        def wait_out(j, p):
            pltpu.make_async_copy(pack_v.at[p], out_ref(j), osem[p]).wait()

        def extract(j, b, p):
            for st in starts:
                iv = idx_v[j, pl.ds(st, 16)]
                row16 = lanes + st
                col0 = (iv & 3) * D

                def kbody(kk, cols):
                    val = plsc.load_gather(rows_v.at[b], [row16, cols])
                    plsc.store_scatter(pack_v.at[p], [row16, cols & (D - 1)],
                                       val)
                    return cols + 1

                lax.fori_loop(0, D, kbody, col0)

        for j in range(2):
            compute_widx(j, j)
            fire_gather(j, j)

        def round_body(r, carry):
            for u in range(NBUF):
                j = r * NBUF + u
                b = u
                p = u % NP
                m2 = (u + 2) % NBUF

                @pl.when(j + 2 < NCHUNK)
                def _():
                    compute_widx(j + 2, m2)
                    fire_gather(m2, m2)

                wait_gather(b)

                @pl.when(j >= NP)
                def _():
                    wait_out(j - NP, p)

                extract(j, b, p)
                fire_out(j, p)
            return carry

        lax.fori_loop(0, NCHUNK // NBUF, round_body, 0)
        for p in range(NP):
            wait_out(NCHUNK - NP + p, p)

    return k


def kernel(table, x):
    V, D = table.shape
    B, L = x.shape
    info = plsc.get_sparse_core_info()
    NC, NS = info.num_cores, info.num_subcores
    NW = NC * NS
    assert B % NW == 0 and V % 4 == 0 and L % 40 == 0
    table_w = table.reshape(V // 4, 4 * D)
    xf = x.reshape(NW, (B // NW) * (L // 40), 40).astype(jnp.int32)
    return _make_kernel(B, L, D, NC, NS)(table_w, xf)
